# baseline (speedup 1.0000x reference)
import functools

import jax
import jax.numpy as jnp
from jax import lax
from jax.experimental import pallas as pl
from jax.experimental.pallas import tpu as pltpu
from jax.experimental.pallas import tpu_sc as plsc

D_MODEL = 64
DPAD = 128
SCALE = 8.0
NC = 2
NS = 16
NW = NC * NS
BATCH = 4096
SEQ = 200
SB = BATCH // NW
VOCAB = 1000000

_mesh = plsc.VectorSubcoreMesh(core_axis_name="c", subcore_axis_name="s")


@functools.partial(
    pl.kernel,
    mesh=_mesh,
    out_type=jax.ShapeDtypeStruct((SEQ, D_MODEL, BATCH), jnp.float32),
    scratch_types=[
        pltpu.VMEM((SEQ, SB), jnp.int32),
        pltpu.VMEM((2, 2 * SB, DPAD), jnp.float32),
        pltpu.VMEM((2, 2, D_MODEL, SB), jnp.float32),
        pltpu.SemaphoreType.DMA((2,)),
        pltpu.SemaphoreType.DMA((2,)),
    ],
    compiler_params=pltpu.CompilerParams(
        use_tc_tiling_on_sc=True, needs_layout_passes=False
    ),
)
def _emb_lookup(xt_hbm, lutp_hbm, out_hbm, idx_all, rows_v, tbuf, sem_g, sem_o):
    wid = lax.axis_index("s") * NC + lax.axis_index("c")
    wbase = wid * SB

    pltpu.sync_copy(xt_hbm.at[pl.ds(0, SEQ), pl.ds(wbase, SB)], idx_all)

    NP = SEQ // 2

    def fire_gather(p, slot):
        for jj in range(2):
            pltpu.async_copy(
                lutp_hbm.at[idx_all.at[2 * p + jj]],
                rows_v.at[slot, pl.ds(jj * SB, SB)],
                sem_g.at[slot],
            )

    def wait_gather(slot):
        for _ in range(2):
            pltpu.make_async_copy(
                lutp_hbm.at[pl.ds(0, SB)],
                rows_v.at[0, pl.ds(0, SB)],
                sem_g.at[slot],
            ).wait()

    def fire_out(p, slot):
        pltpu.async_copy(
            tbuf.at[slot],
            out_hbm.at[pl.ds(2 * p, 2), pl.ds(0, D_MODEL), pl.ds(wbase, SB)],
            sem_o.at[slot],
        )

    def wait_out(slot):
        pltpu.make_async_copy(
            tbuf.at[0],
            out_hbm.at[pl.ds(0, 2), pl.ds(0, D_MODEL), pl.ds(wbase, SB)],
            sem_o.at[slot],
        ).wait()

    iota16 = lax.iota(jnp.int32, 16)
    bases = [(iota16 + 16 * g) * DPAD for g in range(2 * SB // 16)]
    NG = SB // 16

    def transpose_block(slot):
        rflat = rows_v.at[slot].reshape(2 * SB * DPAD)

        @functools.partial(plsc.parallel_loop, 0, D_MODEL, unroll=4)
        def _(d):
            for jj in range(2):
                for g in range(NG):
                    v = plsc.load_gather(rflat, [bases[jj * NG + g] + d])
                    tbuf[slot, jj, d, pl.ds(16 * g, 16)] = v

    def step(p, slot):
        @pl.when(p < NP - 1)
        def _():
            fire_gather(p + 1, 1 - slot)

        wait_gather(slot)

        @pl.when(p >= 2)
        def _():
            wait_out(slot)

        transpose_block(slot)
        fire_out(p, slot)

    fire_gather(0, 0)

    def pair_body(i, c):
        step(2 * i, 0)
        step(2 * i + 1, 1)
        return c

    lax.fori_loop(0, NP // 2, pair_body, 0)
    wait_out(0)
    wait_out(1)


def kernel(x, lut):
    lutp = jnp.pad(lut * SCALE, ((0, 0), (0, DPAD - D_MODEL)))
    outp = _emb_lookup(x.T, lutp)
    return outp.transpose(2, 0, 1)


# speedup vs baseline: 1.0238x; 1.0238x over previous
import functools

import jax
import jax.numpy as jnp
from jax import lax
from jax.experimental import pallas as pl
from jax.experimental.pallas import tpu as pltpu
from jax.experimental.pallas import tpu_sc as plsc

D_MODEL = 64
DPAD = 128
SCALE = 8.0
NC = 2
NS = 16
NW = NC * NS
BATCH = 4096
SEQ = 200
SB = BATCH // NW
VOCAB = 1000000

_mesh = plsc.VectorSubcoreMesh(core_axis_name="c", subcore_axis_name="s")


@functools.partial(
    pl.kernel,
    mesh=_mesh,
    out_type=jax.ShapeDtypeStruct((SEQ, D_MODEL, BATCH), jnp.float32),
    scratch_types=[
        pltpu.VMEM((SEQ, SB), jnp.int32),
        pltpu.VMEM((2, SB, D_MODEL), jnp.float32),
        pltpu.VMEM((2, D_MODEL, SB), jnp.float32),
        pltpu.SemaphoreType.DMA,
        pltpu.SemaphoreType.DMA,
    ],
    compiler_params=pltpu.CompilerParams(
        use_tc_tiling_on_sc=False, needs_layout_passes=False
    ),
)
def _emb_lookup(xt_hbm, lutp_hbm, out_hbm, idx_all, rows_v, tbuf, sem_g, sem_o):
    wid = lax.axis_index("s") * NC + lax.axis_index("c")
    wbase = wid * SB

    pltpu.sync_copy(xt_hbm.at[pl.ds(0, SEQ), pl.ds(wbase, SB)], idx_all)

    def fire_gather(j, slot):
        pltpu.async_copy(
            lutp_hbm.at[idx_all.at[j]],
            rows_v.at[slot],
            sem_g,
        )

    def wait_gather():
        pltpu.make_async_copy(
            lutp_hbm.at[pl.ds(0, SB)],
            rows_v.at[0],
            sem_g,
        ).wait()

    def fire_out(j, slot):
        pltpu.async_copy(
            tbuf.at[slot],
            out_hbm.at[j, pl.ds(0, D_MODEL), pl.ds(wbase, SB)],
            sem_o,
        )

    def wait_out():
        pltpu.make_async_copy(
            tbuf.at[0],
            out_hbm.at[0, pl.ds(0, D_MODEL), pl.ds(wbase, SB)],
            sem_o,
        ).wait()

    iota16 = lax.iota(jnp.int32, 16)
    bases = [(iota16 + 16 * g) * D_MODEL for g in range(SB // 16)]

    def transpose_block(slot):
        rflat = rows_v.at[slot].reshape(SB * D_MODEL)

        @functools.partial(plsc.parallel_loop, 0, D_MODEL, unroll=4)
        def _(d):
            for g in range(SB // 16):
                v = plsc.load_gather(rflat, [bases[g] + d])
                tbuf[slot, d, pl.ds(16 * g, 16)] = v

    def step(j, slot):
        @pl.when(j < SEQ - 1)
        def _():
            fire_gather(j + 1, 1 - slot)

        wait_gather()

        @pl.when(j >= 2)
        def _():
            wait_out()

        transpose_block(slot)
        fire_out(j, slot)

    fire_gather(0, 0)

    def pair_body(i, c):
        step(2 * i, 0)
        step(2 * i + 1, 1)
        return c

    lax.fori_loop(0, SEQ // 2, pair_body, 0)
    wait_out()
    wait_out()


def kernel(x, lut):
    lutp = jnp.pad(lut * SCALE, ((0, 0), (0, DPAD - D_MODEL)))
    lutp2 = lutp.reshape(2 * VOCAB, D_MODEL)
    outp = _emb_lookup(x.T * 2, lutp2)
    return outp.transpose(2, 0, 1)


# speedup vs baseline: 1.4119x; 1.3791x over previous
import functools

import jax
import jax.numpy as jnp
from jax import lax
from jax.experimental import pallas as pl
from jax.experimental.pallas import tpu as pltpu
from jax.experimental.pallas import tpu_sc as plsc

D_MODEL = 64
DPAD = 128
SCALE = 8.0
NC = 2
NS = 16
NW = NC * NS
BATCH = 4096
SEQ = 200
SB = BATCH // NW
VOCAB = 1000000

_mesh = plsc.VectorSubcoreMesh(core_axis_name="c", subcore_axis_name="s")


@functools.partial(
    pl.kernel,
    mesh=_mesh,
    out_type=jax.ShapeDtypeStruct((SEQ, D_MODEL, BATCH), jnp.float32),
    scratch_types=[
        pltpu.VMEM((SEQ, SB), jnp.int32),
        pltpu.VMEM((2, SB, DPAD), jnp.float32),
        pltpu.VMEM((2, D_MODEL, SB), jnp.float32),
        pltpu.SemaphoreType.DMA,
        pltpu.SemaphoreType.DMA,
    ],
    compiler_params=pltpu.CompilerParams(
        use_tc_tiling_on_sc=True, needs_layout_passes=False
    ),
)
def _emb_lookup(xt_hbm, lutp_hbm, out_hbm, idx_all, rows_v, tbuf, sem_g, sem_o):
    wid = lax.axis_index("s") * NC + lax.axis_index("c")
    wbase = wid * SB

    pltpu.sync_copy(xt_hbm.at[pl.ds(0, SEQ), pl.ds(wbase, SB)], idx_all)

    def fire_gather(j, slot):
        pltpu.async_copy(
            lutp_hbm.at[idx_all.at[j]],
            rows_v.at[slot],
            sem_g,
        )

    def wait_gather():
        pltpu.make_async_copy(
            lutp_hbm.at[pl.ds(0, SB)],
            rows_v.at[0],
            sem_g,
        ).wait()

    def fire_out(j, slot):
        pltpu.async_copy(
            tbuf.at[slot],
            out_hbm.at[j, pl.ds(0, D_MODEL), pl.ds(wbase, SB)],
            sem_o,
        )

    def wait_out():
        pltpu.make_async_copy(
            tbuf.at[0],
            out_hbm.at[0, pl.ds(0, D_MODEL), pl.ds(wbase, SB)],
            sem_o,
        ).wait()

    iota16 = lax.iota(jnp.int32, 16)
    bases = [(iota16 + 16 * g) * DPAD for g in range(SB // 16)]

    def transpose_block(slot):
        rflat = rows_v.at[slot].reshape(SB * DPAD)

        @functools.partial(plsc.parallel_loop, 0, D_MODEL, unroll=4)
        def _(d):
            for g in range(SB // 16):
                v = plsc.load_gather(rflat, [bases[g] + d])
                tbuf[slot, d, pl.ds(16 * g, 16)] = v * SCALE

    def step(j, slot):
        @pl.when(j < SEQ - 1)
        def _():
            fire_gather(j + 1, 1 - slot)

        wait_gather()

        @pl.when(j >= 2)
        def _():
            wait_out()

        transpose_block(slot)
        fire_out(j, slot)

    fire_gather(0, 0)

    def pair_body(i, c):
        step(2 * i, 0)
        step(2 * i + 1, 1)
        return c

    lax.fori_loop(0, SEQ // 2, pair_body, 0)
    wait_out()
    wait_out()


def kernel(x, lut):
    lutp = jnp.pad(lut, ((0, 0), (0, DPAD - D_MODEL)))
    outp = _emb_lookup(x.T, lutp)
    return outp.transpose(2, 0, 1)
